# baseline (device time: 200236 ns/iter reference)
import jax
import jax.numpy as jnp
from jax import lax
from jax.experimental import pallas as pl
from jax.experimental.pallas import tpu as pltpu

N_DEV = 8


def kernel(Q, K, V):
    b, s, h, d = Q.shape
    bh = b * h

    bf16 = jnp.bfloat16
    f32 = jnp.float32

    Qt = jnp.transpose(Q, (0, 2, 1, 3)).reshape(bh, s, d).astype(bf16)
    Kt = jnp.transpose(K, (0, 2, 1, 3)).reshape(bh, s, d).astype(bf16)
    Vt = jnp.transpose(V, (0, 2, 1, 3)).reshape(bh, s, d).astype(bf16)
    KV = jnp.concatenate([Kt, Vt], axis=0)

    scale = d ** -0.5

    def body(q_ref, kv_ref, out_ref, kvfull_ref, send_sems, recv_sems):
        my = lax.axis_index("i")
        left = lax.rem(my + N_DEV - 1, N_DEV)
        right = lax.rem(my + 1, N_DEV)

        barrier = pltpu.get_barrier_semaphore()
        for nbr in (left, right):
            pl.semaphore_signal(
                barrier, inc=1,
                device_id=(nbr,), device_id_type=pl.DeviceIdType.MESH,
            )
        pl.semaphore_wait(barrier, 2)

        kvfull_ref[my] = kv_ref[...]

        for t in range(N_DEV - 1):
            slot = lax.rem(my + N_DEV - t, N_DEV)
            rdma = pltpu.make_async_remote_copy(
                src_ref=kvfull_ref.at[slot],
                dst_ref=kvfull_ref.at[slot],
                send_sem=send_sems.at[t],
                recv_sem=recv_sems.at[t],
                device_id=(right,),
                device_id_type=pl.DeviceIdType.MESH,
            )
            rdma.start()
            rdma.wait()

        def compute_one(i, carry):
            q = q_ref[i]
            k_all = jnp.concatenate(
                [kvfull_ref[src, i] for src in range(N_DEV)], axis=0
            )
            v_all = jnp.concatenate(
                [kvfull_ref[src, bh + i] for src in range(N_DEV)], axis=0
            )
            s_mat = lax.dot_general(
                q, k_all, (((1,), (1,)), ((), ())),
                preferred_element_type=f32,
            ) * scale
            m = jnp.max(s_mat, axis=1, keepdims=True)
            p = jnp.exp(s_mat - m)
            p = p / jnp.sum(p, axis=1, keepdims=True)
            o = lax.dot_general(
                p.astype(bf16), v_all, (((1,), (0,)), ((), ())),
                preferred_element_type=f32,
            )
            out_ref[i] = o
            return carry

        lax.fori_loop(0, bh, compute_one, 0)

    out = pl.pallas_call(
        body,
        out_shape=jax.ShapeDtypeStruct((bh, s, d), f32),
        in_specs=[
            pl.BlockSpec(memory_space=pltpu.VMEM),
            pl.BlockSpec(memory_space=pltpu.VMEM),
        ],
        out_specs=pl.BlockSpec(memory_space=pltpu.VMEM),
        scratch_shapes=[
            pltpu.VMEM((N_DEV, 2 * bh, s, d), bf16),
            pltpu.SemaphoreType.DMA((N_DEV - 1,)),
            pltpu.SemaphoreType.DMA((N_DEV - 1,)),
        ],
        compiler_params=pltpu.CompilerParams(collective_id=0),
    )(Qt, KV)

    return out.reshape(b, h, s, d).transpose(0, 2, 1, 3)


# device time: 94775 ns/iter; 2.1128x vs baseline; 2.1128x over previous
import jax
import jax.numpy as jnp
from jax import lax
from jax.experimental import pallas as pl
from jax.experimental.pallas import tpu as pltpu

N_DEV = 8
MASKS = (1, 3, 4)


def kernel(Q, K, V):
    b, s, h, d = Q.shape
    bh = b * h
    rows = 2 * bh

    bf16 = jnp.bfloat16
    f32 = jnp.float32

    Qt = jnp.transpose(Q, (0, 2, 1, 3)).reshape(bh, s, d).astype(bf16)
    Kt = jnp.transpose(K, (0, 2, 1, 3)).reshape(bh, s, d).astype(bf16)
    Vt = jnp.transpose(V, (0, 2, 1, 3)).reshape(bh, s, d).astype(bf16)
    KV = jnp.concatenate([Kt, Vt], axis=0)

    third = rows // 3
    parts = ((0, third + rows % 3), (third + rows % 3, third), (rows - third, third))
    n_rdma = N_DEV - 1

    scale = d ** -0.5

    def body(q_ref, kv_ref, out_ref, kvfull_ref, send_sems, recv_sems):
        my = lax.axis_index("i")
        partners = [my ^ m for m in MASKS]

        barrier = pltpu.get_barrier_semaphore()
        for p in partners:
            pl.semaphore_signal(
                barrier, inc=1,
                device_id=(p,), device_id_type=pl.DeviceIdType.MESH,
            )
        pl.semaphore_wait(barrier, len(partners))

        kvfull_ref[0] = kv_ref[...]

        held = {t: [0] for t in range(3)}
        sem_idx = {t: 0 for t in range(3)}
        prev_round = []
        all_rdmas = []
        for r in range(3):
            for rd in prev_round:
                rd.wait_recv()
            prev_round = []
            for t in range(3):
                m = MASKS[(t + r) % 3]
                lo, n = parts[t]
                for slot in held[t]:
                    rd = pltpu.make_async_remote_copy(
                        src_ref=kvfull_ref.at[slot, pl.ds(lo, n)],
                        dst_ref=kvfull_ref.at[slot ^ m, pl.ds(lo, n)],
                        send_sem=send_sems.at[t, sem_idx[t]],
                        recv_sem=recv_sems.at[t, sem_idx[t]],
                        device_id=(my ^ m,),
                        device_id_type=pl.DeviceIdType.MESH,
                    )
                    rd.start()
                    sem_idx[t] += 1
                    prev_round.append(rd)
                    all_rdmas.append(rd)
                held[t] = held[t] + [slot ^ m for slot in held[t]]
        for rd in prev_round:
            rd.wait_recv()
        for rd in all_rdmas:
            rd.wait_send()

        def compute_one(i, carry):
            q = q_ref[i]
            k_all = jnp.concatenate(
                [kvfull_ref[src, i] for src in range(N_DEV)], axis=0
            )
            v_all = jnp.concatenate(
                [kvfull_ref[src, bh + i] for src in range(N_DEV)], axis=0
            )
            s_mat = lax.dot_general(
                q, k_all, (((1,), (1,)), ((), ())),
                preferred_element_type=f32,
            ) * scale
            m = jnp.max(s_mat, axis=1, keepdims=True)
            p = jnp.exp(s_mat - m)
            p = p / jnp.sum(p, axis=1, keepdims=True)
            o = lax.dot_general(
                p.astype(bf16), v_all, (((1,), (0,)), ((), ())),
                preferred_element_type=f32,
            )
            out_ref[i] = o
            return carry

        lax.fori_loop(0, bh, compute_one, 0)

    out = pl.pallas_call(
        body,
        out_shape=jax.ShapeDtypeStruct((bh, s, d), f32),
        in_specs=[
            pl.BlockSpec(memory_space=pltpu.VMEM),
            pl.BlockSpec(memory_space=pltpu.VMEM),
        ],
        out_specs=pl.BlockSpec(memory_space=pltpu.VMEM),
        scratch_shapes=[
            pltpu.VMEM((N_DEV, rows, s, d), bf16),
            pltpu.SemaphoreType.DMA((3, n_rdma)),
            pltpu.SemaphoreType.DMA((3, n_rdma)),
        ],
        compiler_params=pltpu.CompilerParams(collective_id=0),
    )(Qt, KV)

    return out.reshape(b, h, s, d).transpose(0, 2, 1, 3)


# device time: 86202 ns/iter; 2.3229x vs baseline; 1.0995x over previous
import jax
import jax.numpy as jnp
from jax import lax
from jax.experimental import pallas as pl
from jax.experimental.pallas import tpu as pltpu

N_DEV = 8
MASKS = (1, 3, 4)
LOG2E = 1.4426950408889634


def kernel(Q, K, V):
    b, s, h, d = Q.shape
    bh = b * h

    bf16 = jnp.bfloat16
    f32 = jnp.float32

    Qt = jnp.transpose(Q, (0, 2, 1, 3)).reshape(bh, s, d).astype(bf16)
    Kt = jnp.transpose(K, (0, 2, 1, 3)).reshape(bh, s, d).astype(bf16)
    Vt = jnp.transpose(V, (0, 2, 1, 3)).reshape(bh, s, d).astype(bf16)
    KV = jnp.stack([Kt, Vt], axis=1)

    third = bh // 3
    parts = ((0, bh - 2 * third), (bh - 2 * third, third), (bh - third, third))
    n_rdma = N_DEV - 1

    scale = d ** -0.5

    def body(q_ref, kv_ref, out_ref, kvfull_ref, send_sems, recv_sems):
        my = lax.axis_index("i")
        partners = [my ^ m for m in MASKS]

        barrier = pltpu.get_barrier_semaphore()
        for p in partners:
            pl.semaphore_signal(
                barrier, inc=1,
                device_id=(p,), device_id_type=pl.DeviceIdType.MESH,
            )
        pl.semaphore_wait(barrier, len(partners))

        kvfull_ref[0] = kv_ref[...]

        held = {t: [0] for t in range(3)}
        sem_idx = {t: 0 for t in range(3)}
        prev_round = []
        all_rdmas = []
        for r in range(3):
            for rd in prev_round:
                rd.wait_recv()
            prev_round = []
            for t in range(3):
                m = MASKS[(t + r) % 3]
                lo, n = parts[t]
                for slot in held[t]:
                    rd = pltpu.make_async_remote_copy(
                        src_ref=kvfull_ref.at[slot, pl.ds(lo, n)],
                        dst_ref=kvfull_ref.at[slot ^ m, pl.ds(lo, n)],
                        send_sem=send_sems.at[t, sem_idx[t]],
                        recv_sem=recv_sems.at[t, sem_idx[t]],
                        device_id=(my ^ m,),
                        device_id_type=pl.DeviceIdType.MESH,
                    )
                    rd.start()
                    sem_idx[t] += 1
                    prev_round.append(rd)
                    all_rdmas.append(rd)
                held[t] = held[t] + [slot ^ m for slot in held[t]]
        for rd in prev_round:
            rd.wait_recv()
        for rd in all_rdmas:
            rd.wait_send()

        def compute_one(i, carry):
            q = q_ref[i]
            k_all = jnp.concatenate(
                [kvfull_ref[src, i, 0] for src in range(N_DEV)], axis=0
            )
            v_all = jnp.concatenate(
                [kvfull_ref[src, i, 1] for src in range(N_DEV)], axis=0
            )
            s_mat = lax.dot_general(
                q, k_all, (((1,), (1,)), ((), ())),
                preferred_element_type=f32,
            ) * (scale * LOG2E)
            pf = jnp.exp2(s_mat)
            l = jnp.sum(pf, axis=1, keepdims=True)
            o = lax.dot_general(
                pf.astype(bf16), v_all, (((1,), (0,)), ((), ())),
                preferred_element_type=f32,
            )
            out_ref[i] = o / l
            return carry

        lax.fori_loop(0, bh, compute_one, 0)

    out = pl.pallas_call(
        body,
        out_shape=jax.ShapeDtypeStruct((bh, s, d), f32),
        in_specs=[
            pl.BlockSpec(memory_space=pltpu.VMEM),
            pl.BlockSpec(memory_space=pltpu.VMEM),
        ],
        out_specs=pl.BlockSpec(memory_space=pltpu.VMEM),
        scratch_shapes=[
            pltpu.VMEM((N_DEV, bh, 2, s, d), bf16),
            pltpu.SemaphoreType.DMA((3, n_rdma)),
            pltpu.SemaphoreType.DMA((3, n_rdma)),
        ],
        compiler_params=pltpu.CompilerParams(collective_id=0),
    )(Qt, KV)

    return out.reshape(b, h, s, d).transpose(0, 2, 1, 3)
